# baseline (speedup 1.0000x reference)
import jax
import jax.numpy as jnp
from jax import lax
from jax.experimental import pallas as pl
from jax.experimental.pallas import tpu as pltpu

_B, _T, _N, _F, _H2, _H3, _H1 = 32, 20, 90, 90, 16, 16, 32
_U = _N * _H3
_D = _N * _H2
_G = 4 * _U
_RB = 8
_CB = 1152
_DB = 8
_GB = 4


def _gcn_kernel(fc_a, fc_b, adj_a, adj_b, w_ref, out_ref):
    w = w_ref[...]
    nw = _GB // 2 * _T
    h2s = []
    for fc_ref, adj_ref in ((fc_a, adj_a), (fc_b, adj_b)):
        xw_all = jnp.dot(fc_ref[...].reshape(nw * _N, _F), w,
                         preferred_element_type=jnp.float32)
        for t in range(nw):
            h2s.append(jnp.maximum(
                jnp.dot(adj_ref[t], xw_all[t * _N:(t + 1) * _N],
                        preferred_element_type=jnp.float32),
                0.0))
    out_ref[...] = jnp.stack(h2s).reshape(_GB, _T, _D).astype(jnp.bfloat16)


def _proj_kernel(x_ref, wx_ref, b_ref, out_ref):
    xm = x_ref[...].reshape(_RB * _T, _D)
    z = jnp.dot(xm.astype(jnp.float32), wx_ref[...],
                preferred_element_type=jnp.float32) + b_ref[...]
    out_ref[...] = z.reshape(_RB, _T, _CB).transpose(1, 0, 2).astype(jnp.bfloat16)


def _lstm_kernel(xp_ref, wh_ref, out_ref, h_s, c_s):
    t = pl.program_id(0)

    @pl.when(t == 0)
    def _init():
        h_s[...] = jnp.zeros_like(h_s)
        c_s[...] = jnp.zeros_like(c_s)

    z = xp_ref[0].astype(jnp.float32) + jnp.dot(h_s[...], wh_ref[...],
                            preferred_element_type=jnp.float32)
    i = jax.nn.sigmoid(z[:, :_U])
    f = jax.nn.sigmoid(z[:, _U:2 * _U])
    g = jnp.tanh(z[:, 2 * _U:3 * _U])
    o = jax.nn.sigmoid(z[:, 3 * _U:])
    c = f * c_s[...] + i * g
    h = o * jnp.tanh(c)
    c_s[...] = c
    h_s[...] = h

    @pl.when(t == _T - 1)
    def _emit():
        out_ref[...] = h.reshape(_B, _N, _H3)


def _dec_kernel(h_ref, adj_ref, w_ref, out_ref):
    adj = adj_ref[...]
    w = w_ref[...]
    for k in range(_DB):
        y = jnp.dot(h_ref[k], w, preferred_element_type=jnp.float32)
        h1 = jnp.maximum(
            jnp.dot(adj, y, preferred_element_type=jnp.float32), 0.0)
        r = jnp.maximum(
            lax.dot_general(h1, h1, (((1,), (1,)), ((), ())),
                            preferred_element_type=jnp.float32), 0.0)
        ri = lax.broadcasted_iota(jnp.int32, (_N, _N), 0)
        ci = lax.broadcasted_iota(jnp.int32, (_N, _N), 1)
        out_ref[k] = jnp.where(ri == ci, 1.0, r)


def kernel(sc_features, fc_features, adj_sc, adj_fc, labels, dropout,
           W_fc, Wx, Wh, b_lstm, W_sc):
    x = pl.pallas_call(
        _gcn_kernel,
        grid=(_B // _GB,),
        in_specs=[
            pl.BlockSpec((_GB // 2 * _T, _N, _F), lambda b: (2 * b, 0, 0)),
            pl.BlockSpec((_GB // 2 * _T, _N, _F), lambda b: (2 * b + 1, 0, 0)),
            pl.BlockSpec((_GB // 2 * _T, _N, _N), lambda b: (2 * b, 0, 0)),
            pl.BlockSpec((_GB // 2 * _T, _N, _N), lambda b: (2 * b + 1, 0, 0)),
            pl.BlockSpec((_F, _H2), lambda b: (0, 0)),
        ],
        out_specs=pl.BlockSpec((_GB, _T, _D), lambda b: (b, 0, 0)),
        out_shape=jax.ShapeDtypeStruct((_B, _T, _D), jnp.bfloat16),
        compiler_params=pltpu.CompilerParams(
            dimension_semantics=("parallel",)),
    )(fc_features, fc_features, adj_fc, adj_fc, W_fc)

    xp = pl.pallas_call(
        _proj_kernel,
        grid=(_G // _CB, _B // _RB),
        in_specs=[
            pl.BlockSpec((_RB, _T, _D), lambda j, i: (i, 0, 0)),
            pl.BlockSpec((_D, _CB), lambda j, i: (0, j)),
            pl.BlockSpec((1, _CB), lambda j, i: (0, j)),
        ],
        out_specs=pl.BlockSpec((_T, _RB, _CB), lambda j, i: (0, i, j)),
        out_shape=jax.ShapeDtypeStruct((_T, _B, _G), jnp.bfloat16),
        compiler_params=pltpu.CompilerParams(
            dimension_semantics=("parallel", "parallel")),
    )(x, Wx, b_lstm.reshape(1, _G))

    h = pl.pallas_call(
        _lstm_kernel,
        grid=(_T,),
        in_specs=[
            pl.BlockSpec((1, _B, _G), lambda t: (t, 0, 0)),
            pl.BlockSpec((_U, _G), lambda t: (0, 0)),
        ],
        out_specs=pl.BlockSpec((_B, _N, _H3), lambda t: (0, 0, 0)),
        out_shape=jax.ShapeDtypeStruct((_B, _N, _H3), jnp.float32),
        scratch_shapes=[pltpu.VMEM((_B, _U), jnp.float32),
                        pltpu.VMEM((_B, _U), jnp.float32)],
    )(xp, Wh)

    lstm_h = h
    rec = pl.pallas_call(
        _dec_kernel,
        grid=(_B // _DB,),
        in_specs=[
            pl.BlockSpec((_DB, _N, _H3), lambda i: (i, 0, 0)),
            pl.BlockSpec((_N, _N), lambda i: (0, 0)),
            pl.BlockSpec((_H3, _H1), lambda i: (0, 0)),
        ],
        out_specs=pl.BlockSpec((_DB, _N, _N), lambda i: (i, 0, 0)),
        out_shape=jax.ShapeDtypeStruct((_B, _N, _N), jnp.float32),
        compiler_params=pltpu.CompilerParams(
            dimension_semantics=("parallel",)),
    )(lstm_h, adj_sc, W_sc)
    return rec.reshape(_B, _N * _N)


# speedup vs baseline: 1.0368x; 1.0368x over previous
import jax
import jax.numpy as jnp
from jax import lax
from jax.experimental import pallas as pl
from jax.experimental.pallas import tpu as pltpu

_B, _T, _N, _F, _H2, _H3, _H1 = 32, 20, 90, 90, 16, 16, 32
_U = _N * _H3
_D = _N * _H2
_G = 4 * _U
_RB = 8
_CB = 1152
_DB = 8
_GB = 4


def _gcn_kernel(fc_a, fc_b, adj_a, adj_b, w_ref, out_ref):
    w = w_ref[...]
    h2s = []
    for fc_ref, adj_ref in ((fc_a, adj_a), (fc_b, adj_b)):
        for t in range(_GB // 2 * _T):
            xw = jnp.dot(fc_ref[t], w, preferred_element_type=jnp.float32)
            h2s.append(jnp.maximum(
                jnp.dot(adj_ref[t], xw, preferred_element_type=jnp.float32),
                0.0))
    out_ref[...] = jnp.stack(h2s).reshape(_GB, _T, _D).astype(jnp.bfloat16)


def _proj_kernel(x_ref, wx_ref, b_ref, out_ref):
    xm = x_ref[...].reshape(_RB * _T, _D)
    z = jnp.dot(xm.astype(jnp.float32), wx_ref[...],
                preferred_element_type=jnp.float32) + b_ref[...]
    out_ref[...] = z.reshape(_RB, _T, _CB).transpose(1, 0, 2).astype(jnp.bfloat16)


def _lstm_kernel(xp_ref, wh_ref, adj_ref, wsc_ref, out_ref, h_s, c_s):
    t = pl.program_id(0)

    @pl.when(t == 0)
    def _init():
        h_s[...] = jnp.zeros_like(h_s)
        c_s[...] = jnp.zeros_like(c_s)

    z = xp_ref[0].astype(jnp.float32) + jnp.dot(h_s[...], wh_ref[...],
                            preferred_element_type=jnp.float32)
    i = jax.nn.sigmoid(z[:, :_U])
    f = jax.nn.sigmoid(z[:, _U:2 * _U])
    g = jnp.tanh(z[:, 2 * _U:3 * _U])
    o = jax.nn.sigmoid(z[:, 3 * _U:])
    c = f * c_s[...] + i * g
    h = o * jnp.tanh(c)
    c_s[...] = c
    h_s[...] = h

    @pl.when(t == _T - 1)
    def _emit():
        hr = h.reshape(_B, _N, _H3)
        adj = adj_ref[...]
        wsc = wsc_ref[...]
        ri = lax.broadcasted_iota(jnp.int32, (_N, _N), 0)
        ci = lax.broadcasted_iota(jnp.int32, (_N, _N), 1)
        for k in range(_B):
            y = jnp.dot(hr[k], wsc, preferred_element_type=jnp.float32)
            h1 = jnp.maximum(
                jnp.dot(adj, y, preferred_element_type=jnp.float32), 0.0)
            r = jnp.maximum(
                lax.dot_general(h1, h1, (((1,), (1,)), ((), ())),
                                preferred_element_type=jnp.float32), 0.0)
            out_ref[k] = jnp.where(ri == ci, 1.0, r)


def _dec_kernel(h_ref, adj_ref, w_ref, out_ref):
    adj = adj_ref[...]
    w = w_ref[...]
    for k in range(_DB):
        y = jnp.dot(h_ref[k], w, preferred_element_type=jnp.float32)
        h1 = jnp.maximum(
            jnp.dot(adj, y, preferred_element_type=jnp.float32), 0.0)
        r = jnp.maximum(
            lax.dot_general(h1, h1, (((1,), (1,)), ((), ())),
                            preferred_element_type=jnp.float32), 0.0)
        ri = lax.broadcasted_iota(jnp.int32, (_N, _N), 0)
        ci = lax.broadcasted_iota(jnp.int32, (_N, _N), 1)
        out_ref[k] = jnp.where(ri == ci, 1.0, r)


def kernel(sc_features, fc_features, adj_sc, adj_fc, labels, dropout,
           W_fc, Wx, Wh, b_lstm, W_sc):
    x = pl.pallas_call(
        _gcn_kernel,
        grid=(_B // _GB,),
        in_specs=[
            pl.BlockSpec((_GB // 2 * _T, _N, _F), lambda b: (2 * b, 0, 0)),
            pl.BlockSpec((_GB // 2 * _T, _N, _F), lambda b: (2 * b + 1, 0, 0)),
            pl.BlockSpec((_GB // 2 * _T, _N, _N), lambda b: (2 * b, 0, 0)),
            pl.BlockSpec((_GB // 2 * _T, _N, _N), lambda b: (2 * b + 1, 0, 0)),
            pl.BlockSpec((_F, _H2), lambda b: (0, 0)),
        ],
        out_specs=pl.BlockSpec((_GB, _T, _D), lambda b: (b, 0, 0)),
        out_shape=jax.ShapeDtypeStruct((_B, _T, _D), jnp.bfloat16),
        compiler_params=pltpu.CompilerParams(
            dimension_semantics=("parallel",)),
    )(fc_features, fc_features, adj_fc, adj_fc, W_fc)

    xp = pl.pallas_call(
        _proj_kernel,
        grid=(_G // _CB, _B // _RB),
        in_specs=[
            pl.BlockSpec((_RB, _T, _D), lambda j, i: (i, 0, 0)),
            pl.BlockSpec((_D, _CB), lambda j, i: (0, j)),
            pl.BlockSpec((1, _CB), lambda j, i: (0, j)),
        ],
        out_specs=pl.BlockSpec((_T, _RB, _CB), lambda j, i: (0, i, j)),
        out_shape=jax.ShapeDtypeStruct((_T, _B, _G), jnp.bfloat16),
        compiler_params=pltpu.CompilerParams(
            dimension_semantics=("parallel", "parallel")),
    )(x, Wx, b_lstm.reshape(1, _G))

    rec = pl.pallas_call(
        _lstm_kernel,
        grid=(_T,),
        in_specs=[
            pl.BlockSpec((1, _B, _G), lambda t: (t, 0, 0)),
            pl.BlockSpec((_U, _G), lambda t: (0, 0)),
            pl.BlockSpec((_N, _N), lambda t: (0, 0)),
            pl.BlockSpec((_H3, _H1), lambda t: (0, 0)),
        ],
        out_specs=pl.BlockSpec((_B, _N, _N), lambda t: (0, 0, 0)),
        out_shape=jax.ShapeDtypeStruct((_B, _N, _N), jnp.float32),
        scratch_shapes=[pltpu.VMEM((_B, _U), jnp.float32),
                        pltpu.VMEM((_B, _U), jnp.float32)],
    )(xp, Wh, adj_sc, W_sc)

    return rec.reshape(_B, _N * _N)


# speedup vs baseline: 1.0718x; 1.0338x over previous
import jax
import jax.numpy as jnp
from jax import lax
from jax.experimental import pallas as pl
from jax.experimental.pallas import tpu as pltpu

_B, _T, _N, _F, _H2, _H3, _H1 = 32, 20, 90, 90, 16, 16, 32
_U = _N * _H3
_D = _N * _H2
_G = 4 * _U
_RB = 8
_CB = 1152
_DB = 8
_SP = 2944
_GB = 4


def _gcn_kernel(fc_a, fc_b, adj_a, adj_b, w_ref, out_ref):
    w = w_ref[...]
    h2s = []
    for fc_ref, adj_ref in ((fc_a, adj_a), (fc_b, adj_b)):
        for t in range(_GB // 2 * _T):
            xw = jnp.dot(fc_ref[t], w, preferred_element_type=jnp.float32)
            h2s.append(jnp.maximum(
                jnp.dot(adj_ref[t], xw, preferred_element_type=jnp.float32),
                0.0))
    out_ref[...] = jnp.stack(h2s).reshape(_GB, _T, _D).astype(jnp.bfloat16)


def _proj_kernel(x_ref, wx_ref, b_ref, out_ref):
    xm = x_ref[...].reshape(_RB * _T, _D)
    z = jnp.dot(xm.astype(jnp.float32), wx_ref[...],
                preferred_element_type=jnp.float32) + b_ref[...]
    out_ref[...] = z.reshape(_RB, _T, _CB).transpose(1, 0, 2).astype(jnp.bfloat16)


def _lstm_kernel(xp_ref, wh_ref, adj_ref, wsc_ref, out_ref, h_s, c_s):
    t = pl.program_id(0)

    @pl.when(t == 0)
    def _init():
        h_s[...] = jnp.zeros_like(h_s)
        c_s[...] = jnp.zeros_like(c_s)

    xz = xp_ref[0].astype(jnp.float32)
    hh = h_s[...]
    z1 = xz[:, :_SP] + jnp.dot(hh, wh_ref[:, :_SP],
                               preferred_element_type=jnp.float32)
    i = jax.nn.sigmoid(z1[:, :_U])
    f = jax.nn.sigmoid(z1[:, _U:2 * _U])
    z2 = xz[:, _SP:] + jnp.dot(hh, wh_ref[:, _SP:],
                               preferred_element_type=jnp.float32)
    g = jnp.tanh(jnp.concatenate(
        (z1[:, 2 * _U:], z2[:, :3 * _U - _SP]), axis=1))
    o = jax.nn.sigmoid(z2[:, 3 * _U - _SP:])
    c = f * c_s[...] + i * g
    h = o * jnp.tanh(c)
    c_s[...] = c
    h_s[...] = h

    @pl.when(t == _T - 1)
    def _emit():
        hr = h.reshape(_B, _N, _H3)
        adj = adj_ref[...]
        wsc = wsc_ref[...]
        ri = lax.broadcasted_iota(jnp.int32, (_N, _N), 0)
        ci = lax.broadcasted_iota(jnp.int32, (_N, _N), 1)
        for k in range(_B):
            y = jnp.dot(hr[k], wsc, preferred_element_type=jnp.float32)
            h1 = jnp.maximum(
                jnp.dot(adj, y, preferred_element_type=jnp.float32), 0.0)
            r = jnp.maximum(
                lax.dot_general(h1, h1, (((1,), (1,)), ((), ())),
                                preferred_element_type=jnp.float32), 0.0)
            out_ref[k] = jnp.where(ri == ci, 1.0, r)


def _dec_kernel(h_ref, adj_ref, w_ref, out_ref):
    adj = adj_ref[...]
    w = w_ref[...]
    for k in range(_DB):
        y = jnp.dot(h_ref[k], w, preferred_element_type=jnp.float32)
        h1 = jnp.maximum(
            jnp.dot(adj, y, preferred_element_type=jnp.float32), 0.0)
        r = jnp.maximum(
            lax.dot_general(h1, h1, (((1,), (1,)), ((), ())),
                            preferred_element_type=jnp.float32), 0.0)
        ri = lax.broadcasted_iota(jnp.int32, (_N, _N), 0)
        ci = lax.broadcasted_iota(jnp.int32, (_N, _N), 1)
        out_ref[k] = jnp.where(ri == ci, 1.0, r)


def kernel(sc_features, fc_features, adj_sc, adj_fc, labels, dropout,
           W_fc, Wx, Wh, b_lstm, W_sc):
    x = pl.pallas_call(
        _gcn_kernel,
        grid=(_B // _GB,),
        in_specs=[
            pl.BlockSpec((_GB // 2 * _T, _N, _F), lambda b: (2 * b, 0, 0)),
            pl.BlockSpec((_GB // 2 * _T, _N, _F), lambda b: (2 * b + 1, 0, 0)),
            pl.BlockSpec((_GB // 2 * _T, _N, _N), lambda b: (2 * b, 0, 0)),
            pl.BlockSpec((_GB // 2 * _T, _N, _N), lambda b: (2 * b + 1, 0, 0)),
            pl.BlockSpec((_F, _H2), lambda b: (0, 0)),
        ],
        out_specs=pl.BlockSpec((_GB, _T, _D), lambda b: (b, 0, 0)),
        out_shape=jax.ShapeDtypeStruct((_B, _T, _D), jnp.bfloat16),
        compiler_params=pltpu.CompilerParams(
            dimension_semantics=("parallel",)),
    )(fc_features, fc_features, adj_fc, adj_fc, W_fc)

    xp = pl.pallas_call(
        _proj_kernel,
        grid=(_G // _CB, _B // _RB),
        in_specs=[
            pl.BlockSpec((_RB, _T, _D), lambda j, i: (i, 0, 0)),
            pl.BlockSpec((_D, _CB), lambda j, i: (0, j)),
            pl.BlockSpec((1, _CB), lambda j, i: (0, j)),
        ],
        out_specs=pl.BlockSpec((_T, _RB, _CB), lambda j, i: (0, i, j)),
        out_shape=jax.ShapeDtypeStruct((_T, _B, _G), jnp.bfloat16),
        compiler_params=pltpu.CompilerParams(
            dimension_semantics=("parallel", "parallel")),
    )(x, Wx, b_lstm.reshape(1, _G))

    rec = pl.pallas_call(
        _lstm_kernel,
        grid=(_T,),
        in_specs=[
            pl.BlockSpec((1, _B, _G), lambda t: (t, 0, 0)),
            pl.BlockSpec((_U, _G), lambda t: (0, 0)),
            pl.BlockSpec((_N, _N), lambda t: (0, 0)),
            pl.BlockSpec((_H3, _H1), lambda t: (0, 0)),
        ],
        out_specs=pl.BlockSpec((_B, _N, _N), lambda t: (0, 0, 0)),
        out_shape=jax.ShapeDtypeStruct((_B, _N, _N), jnp.float32),
        scratch_shapes=[pltpu.VMEM((_B, _U), jnp.float32),
                        pltpu.VMEM((_B, _U), jnp.float32)],
    )(xp, Wh, adj_sc, W_sc)

    return rec.reshape(_B, _N * _N)


# speedup vs baseline: 1.1117x; 1.0372x over previous
import jax
import jax.numpy as jnp
from jax import lax
from jax.experimental import pallas as pl
from jax.experimental.pallas import tpu as pltpu

_B, _T, _N, _F, _H2, _H3, _H1 = 32, 20, 90, 90, 16, 16, 32
_U = _N * _H3
_D = _N * _H2
_G = 4 * _U
_RB = 8
_CB = 1152
_DB = 8
_SP = 2944
_GB = 4


def _gcn_proj_kernel(fc_a, fc_b, adj_a, adj_b, w_ref, wx_ref, b_ref,
                     out_ref, x_s):
    j = pl.program_id(0)
    i = pl.program_id(1)

    @pl.when(j == 0)
    def _gcn():
        w = w_ref[...]
        h2s = []
        for fc_ref, adj_ref in ((fc_a, adj_a), (fc_b, adj_b)):
            for t in range(_RB // 2 * _T):
                xw = jnp.dot(fc_ref[t], w, preferred_element_type=jnp.float32)
                h2s.append(jnp.maximum(
                    jnp.dot(adj_ref[t], xw,
                            preferred_element_type=jnp.float32), 0.0))
        x_s[pl.ds(i * _RB, _RB)] = (
            jnp.stack(h2s).reshape(_RB, _T, _D).astype(jnp.bfloat16))

    xm = x_s[pl.ds(i * _RB, _RB)].reshape(_RB * _T, _D)
    z = jnp.dot(xm.astype(jnp.float32), wx_ref[...],
                preferred_element_type=jnp.float32) + b_ref[...]
    out_ref[...] = z.reshape(_RB, _T, _CB).transpose(1, 0, 2).astype(jnp.bfloat16)


def _lstm_kernel(xp_ref, wh_ref, adj_ref, wsc_ref, out_ref, h_s, c_s):
    t = pl.program_id(0)

    @pl.when(t == 0)
    def _init():
        h_s[...] = jnp.zeros_like(h_s)
        c_s[...] = jnp.zeros_like(c_s)

    xz = xp_ref[0].astype(jnp.float32)
    hh = h_s[...]
    z1 = xz[:, :_SP] + jnp.dot(hh, wh_ref[:, :_SP],
                               preferred_element_type=jnp.float32)
    i = jax.nn.sigmoid(z1[:, :_U])
    f = jax.nn.sigmoid(z1[:, _U:2 * _U])
    z2 = xz[:, _SP:] + jnp.dot(hh, wh_ref[:, _SP:],
                               preferred_element_type=jnp.float32)
    g = jnp.tanh(jnp.concatenate(
        (z1[:, 2 * _U:], z2[:, :3 * _U - _SP]), axis=1))
    o = jax.nn.sigmoid(z2[:, 3 * _U - _SP:])
    c = f * c_s[...] + i * g
    h = o * jnp.tanh(c)
    c_s[...] = c
    h_s[...] = h

    @pl.when(t == _T - 1)
    def _emit():
        hr = h.reshape(_B, _N, _H3)
        adj = adj_ref[...]
        wsc = wsc_ref[...]
        ri = lax.broadcasted_iota(jnp.int32, (_N, _N), 0)
        ci = lax.broadcasted_iota(jnp.int32, (_N, _N), 1)
        for k in range(_B):
            y = jnp.dot(hr[k], wsc, preferred_element_type=jnp.float32)
            h1 = jnp.maximum(
                jnp.dot(adj, y, preferred_element_type=jnp.float32), 0.0)
            r = jnp.maximum(
                lax.dot_general(h1, h1, (((1,), (1,)), ((), ())),
                                preferred_element_type=jnp.float32), 0.0)
            out_ref[k] = jnp.where(ri == ci, 1.0, r)


def _dec_kernel(h_ref, adj_ref, w_ref, out_ref):
    adj = adj_ref[...]
    w = w_ref[...]
    for k in range(_DB):
        y = jnp.dot(h_ref[k], w, preferred_element_type=jnp.float32)
        h1 = jnp.maximum(
            jnp.dot(adj, y, preferred_element_type=jnp.float32), 0.0)
        r = jnp.maximum(
            lax.dot_general(h1, h1, (((1,), (1,)), ((), ())),
                            preferred_element_type=jnp.float32), 0.0)
        ri = lax.broadcasted_iota(jnp.int32, (_N, _N), 0)
        ci = lax.broadcasted_iota(jnp.int32, (_N, _N), 1)
        out_ref[k] = jnp.where(ri == ci, 1.0, r)


def kernel(sc_features, fc_features, adj_sc, adj_fc, labels, dropout,
           W_fc, Wx, Wh, b_lstm, W_sc):
    nbh = _B // _RB * 2
    xp = pl.pallas_call(
        _gcn_proj_kernel,
        grid=(_G // _CB, _B // _RB),
        in_specs=[
            pl.BlockSpec((_RB // 2 * _T, _N, _F),
                         lambda j, i: (jnp.where(j == 0, 2 * i, nbh - 2), 0, 0)),
            pl.BlockSpec((_RB // 2 * _T, _N, _F),
                         lambda j, i: (jnp.where(j == 0, 2 * i + 1, nbh - 1), 0, 0)),
            pl.BlockSpec((_RB // 2 * _T, _N, _N),
                         lambda j, i: (jnp.where(j == 0, 2 * i, nbh - 2), 0, 0)),
            pl.BlockSpec((_RB // 2 * _T, _N, _N),
                         lambda j, i: (jnp.where(j == 0, 2 * i + 1, nbh - 1), 0, 0)),
            pl.BlockSpec((_F, _H2), lambda j, i: (0, 0)),
            pl.BlockSpec((_D, _CB), lambda j, i: (0, j)),
            pl.BlockSpec((1, _CB), lambda j, i: (0, j)),
        ],
        out_specs=pl.BlockSpec((_T, _RB, _CB), lambda j, i: (0, i, j)),
        out_shape=jax.ShapeDtypeStruct((_T, _B, _G), jnp.bfloat16),
        scratch_shapes=[pltpu.VMEM((_B, _T, _D), jnp.bfloat16)],
    )(fc_features, fc_features, adj_fc, adj_fc, W_fc, Wx,
      b_lstm.reshape(1, _G))

    rec = pl.pallas_call(
        _lstm_kernel,
        grid=(_T,),
        in_specs=[
            pl.BlockSpec((1, _B, _G), lambda t: (t, 0, 0)),
            pl.BlockSpec((_U, _G), lambda t: (0, 0)),
            pl.BlockSpec((_N, _N), lambda t: (0, 0)),
            pl.BlockSpec((_H3, _H1), lambda t: (0, 0)),
        ],
        out_specs=pl.BlockSpec((_B, _N, _N), lambda t: (0, 0, 0)),
        out_shape=jax.ShapeDtypeStruct((_B, _N, _N), jnp.float32),
        scratch_shapes=[pltpu.VMEM((_B, _U), jnp.float32),
                        pltpu.VMEM((_B, _U), jnp.float32)],
    )(xp, Wh, adj_sc, W_sc)

    return rec.reshape(_B, _N * _N)


# speedup vs baseline: 1.1333x; 1.0195x over previous
import jax
import jax.numpy as jnp
from jax import lax
from jax.experimental import pallas as pl
from jax.experimental.pallas import tpu as pltpu

_B, _T, _N, _F, _H2, _H3, _H1 = 32, 20, 90, 90, 16, 16, 32
_U = _N * _H3
_D = _N * _H2
_G = 4 * _U
_RB = 8
_CB = 1152
_DB = 8
_SP = 2944
_GB = 4


def _gcn_proj_kernel(fc_a, fc_b, adj_a, adj_b, w_ref, wx_ref, b_ref,
                     out_ref, x_s):
    j = pl.program_id(0)
    i = pl.program_id(1)

    @pl.when(j == 0)
    def _gcn():
        w = w_ref[...]
        h2s = []
        for fc_ref, adj_ref in ((fc_a, adj_a), (fc_b, adj_b)):
            for t in range(_RB // 2 * _T):
                xw = jnp.dot(fc_ref[t], w, preferred_element_type=jnp.float32)
                h2s.append(jnp.maximum(
                    jnp.dot(adj_ref[t], xw,
                            preferred_element_type=jnp.float32), 0.0))
        x_s[pl.ds(i * _RB, _RB)] = (
            jnp.stack(h2s).reshape(_RB, _T, _D).astype(jnp.bfloat16))

    xm = x_s[pl.ds(i * _RB, _RB)].reshape(_RB * _T, _D)
    z = jnp.dot(xm.astype(jnp.float32), wx_ref[...],
                preferred_element_type=jnp.float32) + b_ref[...]
    out_ref[...] = z.reshape(_RB, _T, _CB).transpose(1, 0, 2).astype(jnp.bfloat16)


def _lstm_kernel(xp_ref, wh_ref, adj_ref, wsc_ref, out_ref, h_s, c_s):
    t = pl.program_id(0)

    @pl.when(t == 0)
    def _init():
        h_s[...] = jnp.zeros_like(h_s)
        c_s[...] = jnp.zeros_like(c_s)

    xz = xp_ref[0].astype(jnp.float32)
    hh = h_s[...]
    z1 = xz[:, :_SP] + jnp.dot(hh, wh_ref[:, :_SP],
                               preferred_element_type=jnp.float32)
    i = jax.nn.sigmoid(z1[:, :_U])
    f = jax.nn.sigmoid(z1[:, _U:2 * _U])
    z2 = xz[:, _SP:] + jnp.dot(hh, wh_ref[:, _SP:],
                               preferred_element_type=jnp.float32)
    g = jnp.tanh(jnp.concatenate(
        (z1[:, 2 * _U:], z2[:, :3 * _U - _SP]), axis=1))
    o = jax.nn.sigmoid(z2[:, 3 * _U - _SP:])
    c = f * c_s[...] + i * g
    h = o * jnp.tanh(c)
    c_s[...] = c
    h_s[...] = h

    @pl.when(t == _T - 1)
    def _emit():
        hr = h.reshape(_B, _N, _H3)
        wsc = wsc_ref[...]
        yt = jnp.concatenate(
            [jnp.dot(hr[k], wsc, preferred_element_type=jnp.float32)
             for k in range(_B)], axis=1)
        h1_all = jnp.maximum(
            jnp.dot(adj_ref[...], yt, preferred_element_type=jnp.float32),
            0.0)
        ri = lax.broadcasted_iota(jnp.int32, (_N, _N), 0)
        ci = lax.broadcasted_iota(jnp.int32, (_N, _N), 1)
        for k in range(_B):
            h1 = h1_all[:, k * _H1:(k + 1) * _H1]
            r = jnp.maximum(
                lax.dot_general(h1, h1, (((1,), (1,)), ((), ())),
                                preferred_element_type=jnp.float32), 0.0)
            out_ref[k] = jnp.where(ri == ci, 1.0, r)


def _dec_kernel(h_ref, adj_ref, w_ref, out_ref):
    adj = adj_ref[...]
    w = w_ref[...]
    for k in range(_DB):
        y = jnp.dot(h_ref[k], w, preferred_element_type=jnp.float32)
        h1 = jnp.maximum(
            jnp.dot(adj, y, preferred_element_type=jnp.float32), 0.0)
        r = jnp.maximum(
            lax.dot_general(h1, h1, (((1,), (1,)), ((), ())),
                            preferred_element_type=jnp.float32), 0.0)
        ri = lax.broadcasted_iota(jnp.int32, (_N, _N), 0)
        ci = lax.broadcasted_iota(jnp.int32, (_N, _N), 1)
        out_ref[k] = jnp.where(ri == ci, 1.0, r)


def kernel(sc_features, fc_features, adj_sc, adj_fc, labels, dropout,
           W_fc, Wx, Wh, b_lstm, W_sc):
    nbh = _B // _RB * 2
    xp = pl.pallas_call(
        _gcn_proj_kernel,
        grid=(_G // _CB, _B // _RB),
        in_specs=[
            pl.BlockSpec((_RB // 2 * _T, _N, _F),
                         lambda j, i: (jnp.where(j == 0, 2 * i, nbh - 2), 0, 0)),
            pl.BlockSpec((_RB // 2 * _T, _N, _F),
                         lambda j, i: (jnp.where(j == 0, 2 * i + 1, nbh - 1), 0, 0)),
            pl.BlockSpec((_RB // 2 * _T, _N, _N),
                         lambda j, i: (jnp.where(j == 0, 2 * i, nbh - 2), 0, 0)),
            pl.BlockSpec((_RB // 2 * _T, _N, _N),
                         lambda j, i: (jnp.where(j == 0, 2 * i + 1, nbh - 1), 0, 0)),
            pl.BlockSpec((_F, _H2), lambda j, i: (0, 0)),
            pl.BlockSpec((_D, _CB), lambda j, i: (0, j)),
            pl.BlockSpec((1, _CB), lambda j, i: (0, j)),
        ],
        out_specs=pl.BlockSpec((_T, _RB, _CB), lambda j, i: (0, i, j)),
        out_shape=jax.ShapeDtypeStruct((_T, _B, _G), jnp.bfloat16),
        scratch_shapes=[pltpu.VMEM((_B, _T, _D), jnp.bfloat16)],
    )(fc_features, fc_features, adj_fc, adj_fc, W_fc, Wx,
      b_lstm.reshape(1, _G))

    rec = pl.pallas_call(
        _lstm_kernel,
        grid=(_T,),
        in_specs=[
            pl.BlockSpec((1, _B, _G), lambda t: (t, 0, 0)),
            pl.BlockSpec((_U, _G), lambda t: (0, 0)),
            pl.BlockSpec((_N, _N), lambda t: (0, 0)),
            pl.BlockSpec((_H3, _H1), lambda t: (0, 0)),
        ],
        out_specs=pl.BlockSpec((_B, _N, _N), lambda t: (0, 0, 0)),
        out_shape=jax.ShapeDtypeStruct((_B, _N, _N), jnp.float32),
        scratch_shapes=[pltpu.VMEM((_B, _U), jnp.float32),
                        pltpu.VMEM((_B, _U), jnp.float32)],
    )(xp, Wh, adj_sc, W_sc)

    return rec.reshape(_B, _N * _N)


# speedup vs baseline: 1.1707x; 1.0330x over previous
import jax
import jax.numpy as jnp
from jax import lax
from jax.experimental import pallas as pl
from jax.experimental.pallas import tpu as pltpu

_B, _T, _N, _F, _H2, _H3, _H1 = 32, 20, 90, 90, 16, 16, 32
_U = _N * _H3
_D = _N * _H2
_G = 4 * _U
_RB = 8
_CB = 1920
_DB = 8
_SP = 2944
_GB = 4


def _gcn_proj_kernel(fc_a, fc_b, adj_a, adj_b, w_ref, wx_ref, b_ref,
                     out_ref, x_s):
    j = pl.program_id(0)
    i = pl.program_id(1)

    @pl.when(j == 0)
    def _gcn():
        w = w_ref[...]
        h2s = []
        for fc_ref, adj_ref in ((fc_a, adj_a), (fc_b, adj_b)):
            for t in range(_RB // 2 * _T):
                xw = jnp.dot(fc_ref[t], w, preferred_element_type=jnp.float32)
                h2s.append(jnp.maximum(
                    jnp.dot(adj_ref[t], xw,
                            preferred_element_type=jnp.float32), 0.0))
        x_s[pl.ds(i * _RB, _RB)] = (
            jnp.stack(h2s).reshape(_RB, _T, _D).astype(jnp.bfloat16))

    xm = x_s[pl.ds(i * _RB, _RB)].reshape(_RB * _T, _D)
    z = jnp.dot(xm.astype(jnp.float32), wx_ref[...],
                preferred_element_type=jnp.float32) + b_ref[...]
    out_ref[...] = z.reshape(_RB, _T, _CB).transpose(1, 0, 2).astype(jnp.bfloat16)


def _lstm_kernel(xp_ref, wh_ref, adj_ref, wsc_ref, out_ref, h_s, c_s):
    t = pl.program_id(0)

    @pl.when(t == 0)
    def _init():
        h_s[...] = jnp.zeros_like(h_s)
        c_s[...] = jnp.zeros_like(c_s)

    xz = xp_ref[0].astype(jnp.float32)
    hh = h_s[...]
    z1 = xz[:, :_SP] + jnp.dot(hh, wh_ref[:, :_SP],
                               preferred_element_type=jnp.float32)
    i = jax.nn.sigmoid(z1[:, :_U])
    f = jax.nn.sigmoid(z1[:, _U:2 * _U])
    z2 = xz[:, _SP:] + jnp.dot(hh, wh_ref[:, _SP:],
                               preferred_element_type=jnp.float32)
    g = jnp.tanh(jnp.concatenate(
        (z1[:, 2 * _U:], z2[:, :3 * _U - _SP]), axis=1))
    o = jax.nn.sigmoid(z2[:, 3 * _U - _SP:])
    c = f * c_s[...] + i * g
    h = o * jnp.tanh(c)
    c_s[...] = c
    h_s[...] = h

    @pl.when(t == _T - 1)
    def _emit():
        hr = h.reshape(_B, _N, _H3)
        wsc = wsc_ref[...]
        yt = jnp.concatenate(
            [jnp.dot(hr[k], wsc, preferred_element_type=jnp.float32)
             for k in range(_B)], axis=1)
        h1_all = jnp.maximum(
            jnp.dot(adj_ref[...], yt, preferred_element_type=jnp.float32),
            0.0)
        ri = lax.broadcasted_iota(jnp.int32, (_N, _N), 0)
        ci = lax.broadcasted_iota(jnp.int32, (_N, _N), 1)
        for k in range(_B):
            h1 = h1_all[:, k * _H1:(k + 1) * _H1]
            r = jnp.maximum(
                lax.dot_general(h1, h1, (((1,), (1,)), ((), ())),
                                preferred_element_type=jnp.float32), 0.0)
            out_ref[k] = jnp.where(ri == ci, 1.0, r)


def _dec_kernel(h_ref, adj_ref, w_ref, out_ref):
    adj = adj_ref[...]
    w = w_ref[...]
    for k in range(_DB):
        y = jnp.dot(h_ref[k], w, preferred_element_type=jnp.float32)
        h1 = jnp.maximum(
            jnp.dot(adj, y, preferred_element_type=jnp.float32), 0.0)
        r = jnp.maximum(
            lax.dot_general(h1, h1, (((1,), (1,)), ((), ())),
                            preferred_element_type=jnp.float32), 0.0)
        ri = lax.broadcasted_iota(jnp.int32, (_N, _N), 0)
        ci = lax.broadcasted_iota(jnp.int32, (_N, _N), 1)
        out_ref[k] = jnp.where(ri == ci, 1.0, r)


def kernel(sc_features, fc_features, adj_sc, adj_fc, labels, dropout,
           W_fc, Wx, Wh, b_lstm, W_sc):
    nbh = _B // _RB * 2
    xp = pl.pallas_call(
        _gcn_proj_kernel,
        grid=(_G // _CB, _B // _RB),
        in_specs=[
            pl.BlockSpec((_RB // 2 * _T, _N, _F),
                         lambda j, i: (jnp.where(j == 0, 2 * i, nbh - 2), 0, 0)),
            pl.BlockSpec((_RB // 2 * _T, _N, _F),
                         lambda j, i: (jnp.where(j == 0, 2 * i + 1, nbh - 1), 0, 0)),
            pl.BlockSpec((_RB // 2 * _T, _N, _N),
                         lambda j, i: (jnp.where(j == 0, 2 * i, nbh - 2), 0, 0)),
            pl.BlockSpec((_RB // 2 * _T, _N, _N),
                         lambda j, i: (jnp.where(j == 0, 2 * i + 1, nbh - 1), 0, 0)),
            pl.BlockSpec((_F, _H2), lambda j, i: (0, 0)),
            pl.BlockSpec((_D, _CB), lambda j, i: (0, j)),
            pl.BlockSpec((1, _CB), lambda j, i: (0, j)),
        ],
        out_specs=pl.BlockSpec((_T, _RB, _CB), lambda j, i: (0, i, j)),
        out_shape=jax.ShapeDtypeStruct((_T, _B, _G), jnp.bfloat16),
        scratch_shapes=[pltpu.VMEM((_B, _T, _D), jnp.bfloat16)],
    )(fc_features, fc_features, adj_fc, adj_fc, W_fc, Wx,
      b_lstm.reshape(1, _G))

    rec = pl.pallas_call(
        _lstm_kernel,
        grid=(_T,),
        in_specs=[
            pl.BlockSpec((1, _B, _G), lambda t: (t, 0, 0)),
            pl.BlockSpec((_U, _G), lambda t: (0, 0)),
            pl.BlockSpec((_N, _N), lambda t: (0, 0)),
            pl.BlockSpec((_H3, _H1), lambda t: (0, 0)),
        ],
        out_specs=pl.BlockSpec((_B, _N, _N), lambda t: (0, 0, 0)),
        out_shape=jax.ShapeDtypeStruct((_B, _N, _N), jnp.float32),
        scratch_shapes=[pltpu.VMEM((_B, _U), jnp.float32),
                        pltpu.VMEM((_B, _U), jnp.float32)],
    )(xp, Wh, adj_sc, W_sc)

    return rec.reshape(_B, _N * _N)
